# baseline (speedup 1.0000x reference)
import functools

import jax
import jax.numpy as jnp
from jax import lax
from jax.experimental import pallas as pl
from jax.experimental.pallas import tpu as pltpu
from jax.experimental.pallas import tpu_sc as plsc

_BN_SCALE = float(1.0 / (1.0 + 1e-5) ** 0.5)


def _lane_perm(x, idx):
  return lax.gather(
      x, idx[:, None],
      dimension_numbers=lax.GatherDimensionNumbers(
          offset_dims=(), collapsed_slice_dims=(0,), start_index_map=(0,)),
      slice_sizes=(1,),
      mode=lax.GatherScatterMode.PROMISE_IN_BOUNDS)


_NC = 2
_NS = 16
_NW = _NC * _NS
_L = 16
_CH = 64
_NBUF = 4


@functools.lru_cache(maxsize=None)
def _build(B, D):
  b_per_w = B // _NW
  nch = b_per_w // _CH
  nk = D // _L
  mesh = plsc.VectorSubcoreMesh(
      core_axis_name="c", subcore_axis_name="s",
      num_cores=_NC, num_subcores=_NS)

  @functools.partial(
      pl.kernel,
      out_type=jax.ShapeDtypeStruct((B,), jnp.float32),
      mesh=mesh,
      compiler_params=pltpu.CompilerParams(needs_layout_passes=False,
                                           skip_device_barrier=True),
      scratch_types=[
          pltpu.VMEM((nch, _CH), jnp.int32),
          pltpu.VMEM((nch, _CH), jnp.int32),
          pltpu.VMEM((_NBUF, _CH, D), jnp.float32),
          pltpu.VMEM((_NBUF, _CH, D), jnp.float32),
          pltpu.VMEM((b_per_w,), jnp.float32),
          pltpu.VMEM((b_per_w,), jnp.float32),
          pltpu.VMEM((b_per_w,), jnp.float32),
          pltpu.VMEM((_L,), jnp.float32),
          pltpu.VMEM((D,), jnp.float32),
          pltpu.VMEM((D,), jnp.float32),
          pltpu.VMEM((D,), jnp.float32),
          pltpu.VMEM((D,), jnp.float32),
          pltpu.SemaphoreType.DMA,
          pltpu.SemaphoreType.DMA,
          pltpu.SemaphoreType.DMA,
          pltpu.SemaphoreType.DMA,
          pltpu.SemaphoreType.DMA,
          pltpu.SemaphoreType.DMA,
          pltpu.SemaphoreType.DMA,
          pltpu.SemaphoreType.DMA,
          pltpu.SemaphoreType.DMA,
      ],
  )
  def sc_kernel(idx_u_hbm, idx_i_hbm, ut_hbm, it_hbm,
                gu_hbm, bu_hbm, gi_hbm, bi_hbm, gb_hbm, ub_hbm, ib_hbm,
                out_hbm,
                idx_uv, idx_iv, ubuf, ibuf, outv, ubv, ibv, gbv,
                guv, buv, giv, biv,
                sem_u0, sem_u1, sem_u2, sem_u3,
                sem_i0, sem_i1, sem_i2, sem_i3, sem_x):
    wid = lax.axis_index("s") * _NC + lax.axis_index("c")
    base_row = wid * b_per_w

    idx_cps = []
    for c in range(nch):
      src = idx_u_hbm.at[pl.ds(base_row + c * _CH, _CH)]
      idx_cps.append(pltpu.async_copy(src, idx_uv.at[c], sem_x))
      src = idx_i_hbm.at[pl.ds(base_row + c * _CH, _CH)]
      idx_cps.append(pltpu.async_copy(src, idx_iv.at[c], sem_x))
    for cp in idx_cps:
      cp.wait()

    sems_u = [sem_u0, sem_u1, sem_u2, sem_u3]
    sems_i = [sem_i0, sem_i1, sem_i2, sem_i3]

    def start(c):
      slot = c % _NBUF
      hu = pltpu.async_copy(ut_hbm.at[idx_uv.at[c]], ubuf.at[slot],
                            sems_u[slot])
      hi = pltpu.async_copy(it_hbm.at[idx_iv.at[c]], ibuf.at[slot],
                            sems_i[slot])
      return hu, hi

    for c in range(min(_NBUF, nch)):
      start(c)

    pltpu.sync_copy(gu_hbm, guv)
    pltpu.sync_copy(bu_hbm, buv)
    pltpu.sync_copy(gi_hbm, giv)
    pltpu.sync_copy(bi_hbm, biv)
    pltpu.sync_copy(ub_hbm.at[pl.ds(base_row, b_per_w)], ubv)
    pltpu.sync_copy(ib_hbm.at[pl.ds(base_row, b_per_w)], ibv)
    pltpu.sync_copy(gb_hbm, gbv)

    gus = [guv[pl.ds(k * _L, _L)] * _BN_SCALE for k in range(nk)]
    bus = [buv[pl.ds(k * _L, _L)] for k in range(nk)]
    gis = [giv[pl.ds(k * _L, _L)] * _BN_SCALE for k in range(nk)]
    bis = [biv[pl.ds(k * _L, _L)] for k in range(nk)]

    lanes = lax.iota(jnp.int32, _L)
    idx_hi = [(lanes + (1 << l)) & (_L - 1) for l in range(4)]
    idx_lo = [(lanes - (1 << l)) & (_L - 1) for l in range(4)]
    sel_lo = [(lanes & ((2 << l) - 1)) < (1 << l) for l in range(4)]

    gb = gbv[pl.ds(0, _L)]
    gpc = _CH // _L

    def _group(g, carry):
      for c in range(nch):
        @pl.when(g == c * gpc)
        def _(c=c):
          slot = c % _NBUF
          pltpu.make_async_copy(ut_hbm.at[idx_uv.at[c]], ubuf.at[slot],
                                sems_u[slot]).wait()
          pltpu.make_async_copy(it_hbm.at[idx_iv.at[c]], ibuf.at[slot],
                                sems_i[slot]).wait()

      slot = (g // gpc) % _NBUF

      pending = []

      def merge(x, y, lvl):
        zx = x + _lane_perm(x, idx_hi[lvl])
        zy = y + _lane_perm(y, idx_lo[lvl])
        return jnp.where(sel_lo[lvl], zx, zy)

      rbase = (g % gpc) * _L
      for j in range(_L):
        r = rbase + j
        ps = []
        for k in range(nk):
          sl = pl.ds(k * _L, _L)
          u = ubuf[slot, r, sl]
          i = ibuf[slot, r, sl]
          ue = u * gus[k] + bus[k]
          ie = i * gis[k] + bis[k]
          ps.append(ue * ie)
        while len(ps) > 1:
          ps = [ps[a] + ps[a + 1] for a in range(0, len(ps), 2)]
        node = (0, ps[0])
        while pending and pending[-1][0] == node[0]:
          prev = pending.pop()
          node = (node[0] + 1, merge(prev[1], node[1], node[0]))
        pending.append(node)

      osl = pl.ds(g * _L, _L)
      outv[osl] = pending[0][1] + ubv[osl] + ibv[osl] + gb

      for c in range(nch - _NBUF):
        @pl.when(g == (c + 1) * gpc - 1)
        def _(c=c):
          start(c + _NBUF)
      return carry

    lax.fori_loop(0, nch * gpc, _group, 0)
    pltpu.sync_copy(outv, out_hbm.at[pl.ds(base_row, b_per_w)])

  return sc_kernel


def kernel(users_ids, items_ids, user_table, item_table,
           user_bn_gamma, user_bn_beta, item_bn_gamma, item_bn_beta,
           global_bias, user_bias, item_bias):
  B = users_ids.shape[0]
  D = user_table.shape[1]
  idx_u = users_ids.astype(jnp.int32)
  idx_i = items_ids.astype(jnp.int32)
  gb = jnp.broadcast_to(global_bias.astype(jnp.float32).reshape(1), (_L,))
  return _build(B, D)(idx_u, idx_i, user_table, item_table,
                      user_bn_gamma, user_bn_beta, item_bn_gamma,
                      item_bn_beta, gb, user_bias, item_bias)


# speedup vs baseline: 1.0021x; 1.0021x over previous
import functools

import jax
import jax.numpy as jnp
from jax import lax
from jax.experimental import pallas as pl
from jax.experimental.pallas import tpu as pltpu
from jax.experimental.pallas import tpu_sc as plsc

_BN_SCALE = float(1.0 / (1.0 + 1e-5) ** 0.5)


def _lane_perm(x, idx):
  return lax.gather(
      x, idx[:, None],
      dimension_numbers=lax.GatherDimensionNumbers(
          offset_dims=(), collapsed_slice_dims=(0,), start_index_map=(0,)),
      slice_sizes=(1,),
      mode=lax.GatherScatterMode.PROMISE_IN_BOUNDS)


_NC = 2
_NS = 16
_NW = _NC * _NS
_L = 16
_CH = 128
_NBUF = 3


@functools.lru_cache(maxsize=None)
def _build(B, D):
  b_per_w = B // _NW
  nch = b_per_w // _CH
  nk = D // _L
  mesh = plsc.VectorSubcoreMesh(
      core_axis_name="c", subcore_axis_name="s",
      num_cores=_NC, num_subcores=_NS)

  @functools.partial(
      pl.kernel,
      out_type=jax.ShapeDtypeStruct((B,), jnp.float32),
      mesh=mesh,
      compiler_params=pltpu.CompilerParams(needs_layout_passes=False,
                                           skip_device_barrier=True),
      scratch_types=[
          pltpu.VMEM((nch, _CH), jnp.int32),
          pltpu.VMEM((nch, _CH), jnp.int32),
          pltpu.VMEM((_NBUF, _CH, D), jnp.float32),
          pltpu.VMEM((_NBUF, _CH, D), jnp.float32),
          pltpu.VMEM((b_per_w,), jnp.float32),
          pltpu.VMEM((b_per_w,), jnp.float32),
          pltpu.VMEM((b_per_w,), jnp.float32),
          pltpu.VMEM((_L,), jnp.float32),
          pltpu.VMEM((D,), jnp.float32),
          pltpu.VMEM((D,), jnp.float32),
          pltpu.VMEM((D,), jnp.float32),
          pltpu.VMEM((D,), jnp.float32),
          pltpu.SemaphoreType.DMA,
          pltpu.SemaphoreType.DMA,
          pltpu.SemaphoreType.DMA,
          pltpu.SemaphoreType.DMA,
          pltpu.SemaphoreType.DMA,
          pltpu.SemaphoreType.DMA,
          pltpu.SemaphoreType.DMA,
          pltpu.SemaphoreType.DMA,
          pltpu.SemaphoreType.DMA,
      ],
  )
  def sc_kernel(idx_u_hbm, idx_i_hbm, ut_hbm, it_hbm,
                gu_hbm, bu_hbm, gi_hbm, bi_hbm, gb_hbm, ub_hbm, ib_hbm,
                out_hbm,
                idx_uv, idx_iv, ubuf, ibuf, outv, ubv, ibv, gbv,
                guv, buv, giv, biv,
                sem_u0, sem_u1, sem_u2, sem_u3,
                sem_i0, sem_i1, sem_i2, sem_i3, sem_x):
    wid = lax.axis_index("s") * _NC + lax.axis_index("c")
    base_row = wid * b_per_w

    idx_cps = []
    for c in range(nch):
      src = idx_u_hbm.at[pl.ds(base_row + c * _CH, _CH)]
      idx_cps.append(pltpu.async_copy(src, idx_uv.at[c], sem_x))
      src = idx_i_hbm.at[pl.ds(base_row + c * _CH, _CH)]
      idx_cps.append(pltpu.async_copy(src, idx_iv.at[c], sem_x))
    for cp in idx_cps:
      cp.wait()

    sems_u = [sem_u0, sem_u1, sem_u2, sem_u3]
    sems_i = [sem_i0, sem_i1, sem_i2, sem_i3]

    def start(c):
      slot = c % _NBUF
      hu = pltpu.async_copy(ut_hbm.at[idx_uv.at[c]], ubuf.at[slot],
                            sems_u[slot])
      hi = pltpu.async_copy(it_hbm.at[idx_iv.at[c]], ibuf.at[slot],
                            sems_i[slot])
      return hu, hi

    for c in range(min(_NBUF, nch)):
      start(c)

    pltpu.sync_copy(gu_hbm, guv)
    pltpu.sync_copy(bu_hbm, buv)
    pltpu.sync_copy(gi_hbm, giv)
    pltpu.sync_copy(bi_hbm, biv)
    pltpu.sync_copy(ub_hbm.at[pl.ds(base_row, b_per_w)], ubv)
    pltpu.sync_copy(ib_hbm.at[pl.ds(base_row, b_per_w)], ibv)
    pltpu.sync_copy(gb_hbm, gbv)

    gus = [guv[pl.ds(k * _L, _L)] * _BN_SCALE for k in range(nk)]
    bus = [buv[pl.ds(k * _L, _L)] for k in range(nk)]
    gis = [giv[pl.ds(k * _L, _L)] * _BN_SCALE for k in range(nk)]
    bis = [biv[pl.ds(k * _L, _L)] for k in range(nk)]

    lanes = lax.iota(jnp.int32, _L)
    idx_hi = [(lanes + (1 << l)) & (_L - 1) for l in range(4)]
    idx_lo = [(lanes - (1 << l)) & (_L - 1) for l in range(4)]
    sel_lo = [(lanes & ((2 << l) - 1)) < (1 << l) for l in range(4)]

    gb = gbv[pl.ds(0, _L)]
    gpc = _CH // _L

    def _group(g, carry):
      for c in range(nch):
        @pl.when(g == c * gpc)
        def _(c=c):
          slot = c % _NBUF
          pltpu.make_async_copy(ut_hbm.at[idx_uv.at[c]], ubuf.at[slot],
                                sems_u[slot]).wait()
          pltpu.make_async_copy(it_hbm.at[idx_iv.at[c]], ibuf.at[slot],
                                sems_i[slot]).wait()

      slot = (g // gpc) % _NBUF

      pending = []

      def merge(x, y, lvl):
        zx = x + _lane_perm(x, idx_hi[lvl])
        zy = y + _lane_perm(y, idx_lo[lvl])
        return jnp.where(sel_lo[lvl], zx, zy)

      rbase = (g % gpc) * _L
      for j in range(_L):
        r = rbase + j
        ps = []
        for k in range(nk):
          sl = pl.ds(k * _L, _L)
          u = ubuf[slot, r, sl]
          i = ibuf[slot, r, sl]
          ue = u * gus[k] + bus[k]
          ie = i * gis[k] + bis[k]
          ps.append(ue * ie)
        while len(ps) > 1:
          ps = [ps[a] + ps[a + 1] for a in range(0, len(ps), 2)]
        node = (0, ps[0])
        while pending and pending[-1][0] == node[0]:
          prev = pending.pop()
          node = (node[0] + 1, merge(prev[1], node[1], node[0]))
        pending.append(node)

      osl = pl.ds(g * _L, _L)
      outv[osl] = pending[0][1] + ubv[osl] + ibv[osl] + gb

      for c in range(nch - _NBUF):
        @pl.when(g == (c + 1) * gpc - 1)
        def _(c=c):
          start(c + _NBUF)
      return carry

    lax.fori_loop(0, nch * gpc, _group, 0)
    pltpu.sync_copy(outv, out_hbm.at[pl.ds(base_row, b_per_w)])

  return sc_kernel


def kernel(users_ids, items_ids, user_table, item_table,
           user_bn_gamma, user_bn_beta, item_bn_gamma, item_bn_beta,
           global_bias, user_bias, item_bias):
  B = users_ids.shape[0]
  D = user_table.shape[1]
  idx_u = users_ids.astype(jnp.int32)
  idx_i = items_ids.astype(jnp.int32)
  gb = jnp.broadcast_to(global_bias.astype(jnp.float32).reshape(1), (_L,))
  return _build(B, D)(idx_u, idx_i, user_table, item_table,
                      user_bn_gamma, user_bn_beta, item_bn_gamma,
                      item_bn_beta, gb, user_bias, item_bias)
